# baseline (speedup 1.0000x reference)
import jax
import jax.numpy as jnp
import numpy as np
from jax.experimental import pallas as pl
from jax.experimental.pallas import tpu as pltpu

_B = 128
_V = 100000
_W = 2048
_NB = pl.cdiv(_V, _W)
_RG = 64
_NEG_INF = np.float32(-np.inf)
_TINY = np.float32(1.1754943508222875e-38)
_INT_MAX = np.int32(2**31 - 1)


def _threefry_bits(flat_i32):
    ks0 = np.uint32(0)
    ks1 = np.uint32(42)
    ks2 = ks0 ^ ks1 ^ np.uint32(0x1BD11BDA)
    rot = ((13, 15, 26, 6), (17, 29, 16, 24))
    x1 = flat_i32.astype(jnp.uint32)
    x0 = jnp.zeros_like(x1) + ks0
    x1 = x1 + ks1
    ks = (ks0, ks1, ks2)
    for r in range(5):
        for rr in rot[r % 2]:
            x0 = x0 + x1
            x1 = (x1 << np.uint32(rr)) | (x1 >> np.uint32(32 - rr))
            x1 = x1 ^ x0
        x0 = x0 + ks[(r + 1) % 3]
        x1 = x1 + ks[(r + 2) % 3] + np.uint32(r + 1)
    return x0 ^ x1


def _body(logits_ref, act_ref, lp_ref, mode_ref, samp_ref,
          m_ref, s_ref, g_ref, av_ref, ai_ref, sv_ref, si_ref):
    rg = pl.program_id(0)
    j = pl.program_id(1)

    @pl.when(j == 0)
    def _init():
        m_ref[...] = jnp.full((_RG, 1), _NEG_INF, jnp.float32)
        s_ref[...] = jnp.zeros((_RG, 1), jnp.float32)
        g_ref[...] = jnp.zeros((_RG, 1), jnp.float32)
        av_ref[...] = jnp.full((_RG, 1), _NEG_INF, jnp.float32)
        ai_ref[...] = jnp.zeros((_RG, 1), jnp.int32)
        sv_ref[...] = jnp.full((_RG, 1), _NEG_INF, jnp.float32)
        si_ref[...] = jnp.zeros((_RG, 1), jnp.int32)

    x = logits_ref[...]
    col = j * _W + jax.lax.broadcasted_iota(jnp.int32, (_RG, _W), 1)
    valid = col < _V
    xm = jnp.where(valid, x, _NEG_INF)

    row = rg * _RG + jax.lax.broadcasted_iota(jnp.int32, (_RG, _W), 0)
    bits = _threefry_bits(row * _V + col)
    fbits = (bits >> np.uint32(9)) | np.uint32(0x3F800000)
    floats = jax.lax.bitcast_convert_type(fbits, jnp.float32) - np.float32(1.0)
    u = jnp.maximum(_TINY, floats + _TINY)
    gum = -jnp.log(-jnp.log(u))
    phi = jnp.where(valid, x + gum, _NEG_INF)

    bmax = jnp.max(xm, axis=1, keepdims=True)
    m_old = m_ref[...]
    m_new = jnp.maximum(m_old, bmax)
    s_ref[...] = (s_ref[...] * jnp.exp(m_old - m_new)
                  + jnp.sum(jnp.exp(xm - m_new), axis=1, keepdims=True))
    m_ref[...] = m_new

    act = act_ref[...]
    g_ref[...] += jnp.sum(jnp.where(col == act, x, 0.0), axis=1, keepdims=True)

    bidx = jnp.min(jnp.where(xm == bmax, col, _INT_MAX), axis=1, keepdims=True)
    better = bmax > av_ref[...]
    av_ref[...] = jnp.where(better, bmax, av_ref[...])
    ai_ref[...] = jnp.where(better, bidx, ai_ref[...])

    pmax = jnp.max(phi, axis=1, keepdims=True)
    pidx = jnp.min(jnp.where(phi == pmax, col, _INT_MAX), axis=1, keepdims=True)
    sbetter = pmax > sv_ref[...]
    sv_ref[...] = jnp.where(sbetter, pmax, sv_ref[...])
    si_ref[...] = jnp.where(sbetter, pidx, si_ref[...])

    @pl.when(j == _NB - 1)
    def _fin():
        lp_ref[...] = g_ref[...] - (m_ref[...] + jnp.log(s_ref[...]))
        mode_ref[...] = ai_ref[...]
        samp_ref[...] = si_ref[...]


_GRID_SPEC = dict(
    grid=(_B // _RG, _NB),
    in_specs=[
        pl.BlockSpec((_RG, _W), lambda rg, j: (rg, j)),
        pl.BlockSpec((_RG, 1), lambda rg, j: (rg, 0)),
    ],
    out_specs=[
        pl.BlockSpec((_RG, 1), lambda rg, j: (rg, 0)),
        pl.BlockSpec((_RG, 1), lambda rg, j: (rg, 0)),
        pl.BlockSpec((_RG, 1), lambda rg, j: (rg, 0)),
    ],
    out_shape=[
        jax.ShapeDtypeStruct((_B, 1), jnp.float32),
        jax.ShapeDtypeStruct((_B, 1), jnp.int32),
        jax.ShapeDtypeStruct((_B, 1), jnp.int32),
    ],
    scratch_shapes=[
        pltpu.VMEM((_RG, 1), jnp.float32),
        pltpu.VMEM((_RG, 1), jnp.float32),
        pltpu.VMEM((_RG, 1), jnp.float32),
        pltpu.VMEM((_RG, 1), jnp.float32),
        pltpu.VMEM((_RG, 1), jnp.int32),
        pltpu.VMEM((_RG, 1), jnp.float32),
        pltpu.VMEM((_RG, 1), jnp.int32),
    ],
)


def kernel(logits, actions):
    lp, mode, samp = pl.pallas_call(
        _body,
        compiler_params=pltpu.CompilerParams(
            dimension_semantics=("parallel", "arbitrary")),
        **_GRID_SPEC,
    )(logits, actions)
    return (lp, mode, samp)
